# baseline (speedup 1.0000x reference)
import functools

import jax
import jax.numpy as jnp
from jax import lax
from jax.experimental import pallas as pl
from jax.experimental.pallas import tpu as pltpu
from jax.experimental.pallas import tpu_sc as plsc


def _sc_gather(table, ids, n, r):
    info = plsc.get_sparse_core_info()
    nc, ns = info.num_cores, info.num_subcores
    nw = nc * ns
    n_per_w = n // nw

    mesh = plsc.VectorSubcoreMesh(core_axis_name="c", subcore_axis_name="s")

    @functools.partial(
        pl.kernel,
        mesh=mesh,
        out_type=jax.ShapeDtypeStruct((n, r), jnp.float32),
        scratch_types=[
            pltpu.VMEM((n_per_w,), jnp.int32),
            pltpu.VMEM((n_per_w, r), jnp.float32),
            pltpu.SemaphoreType.DMA,
            pltpu.SemaphoreType.DMA,
        ],
    )
    def gather_rows(table_hbm, idx_hbm, out_hbm, idx_v, rows_v, sem, osem):
        wid = lax.axis_index("s") * nc + lax.axis_index("c")
        base = wid * n_per_w
        pltpu.sync_copy(idx_hbm.at[pl.ds(base, n_per_w)], idx_v)

        def body(jc, _):
            vec = idx_v[pl.ds(jc * 16, 16)]
            for lane in range(16):
                rid = vec[lane]
                pltpu.async_copy(
                    table_hbm.at[pl.ds(rid, 1)],
                    rows_v.at[pl.ds(jc * 16 + lane, 1)],
                    sem,
                )
            return 0

        lax.fori_loop(0, n_per_w // 16, body, 0)
        out_slice = out_hbm.at[pl.ds(base, n_per_w)]
        pltpu.make_async_copy(out_slice, rows_v, sem).wait()
        pltpu.async_copy(rows_v, out_slice, osem).wait()

    return gather_rows(table, ids)


def kernel(input_ids, input_states, lora_A, lora_B_w):
    b, s = input_ids.shape
    h = input_states.shape[-1]
    r = lora_A.shape[1]
    n = b * s

    ids = input_ids.reshape(n).astype(jnp.int32)
    gathered = _sc_gather(lora_A, ids, n, r)

    x2d = input_states.reshape(n, h)
    blk = 512

    def tc_body(a_ref, x_ref, w_ref, o_ref):
        prj = lax.dot_general(
            a_ref[...],
            w_ref[...],
            dimension_numbers=(((1,), (1,)), ((), ())),
            preferred_element_type=jnp.float32,
        )
        o_ref[...] = x_ref[...] + prj

    out2d = pl.pallas_call(
        tc_body,
        grid=(n // blk,),
        in_specs=[
            pl.BlockSpec((blk, r), lambda i: (i, 0)),
            pl.BlockSpec((blk, h), lambda i: (i, 0)),
            pl.BlockSpec((h, r), lambda i: (0, 0)),
        ],
        out_specs=pl.BlockSpec((blk, h), lambda i: (i, 0)),
        out_shape=jax.ShapeDtypeStruct((n, h), jnp.float32),
    )(gathered, x2d, lora_B_w)

    return out2d.reshape(b, s, h)
